# baseline (speedup 1.0000x reference)
import functools

import jax
import jax.numpy as jnp
from jax import lax
from jax.experimental import pallas as pl
from jax.experimental.pallas import tpu as pltpu
from jax.experimental.pallas import tpu_sc as plsc

_B = 32768
_D = 2048
_E = 64
_BLK = 1024

_INFO = plsc.get_sparse_core_info()
_NW = _INFO.num_cores * _INFO.num_subcores
_TPW = _B // _NW


def _router_body(rv_ref, emb_ref, scores_t_ref, w_ref):
    @pl.when(pl.program_id(0) == 0)
    def _prep():
        emb = emb_ref[...]
        norms = jnp.clip(jnp.sqrt(jnp.sum(emb * emb, axis=1, keepdims=True)), 1e-12)
        w_ref[...] = emb / norms

    scores_t_ref[...] = jax.lax.dot_general(
        w_ref[...], rv_ref[...],
        dimension_numbers=(((1,), (1,)), ((), ())),
        preferred_element_type=jnp.float32,
    )


def _scores_t(route_vec, expert_embeddings):
    return pl.pallas_call(
        _router_body,
        grid=(_B // _BLK,),
        in_specs=[
            pl.BlockSpec((_BLK, _D), lambda i: (i, 0)),
            pl.BlockSpec((_E, _D), lambda i: (0, 0)),
        ],
        out_specs=pl.BlockSpec((_E, _BLK), lambda i: (0, i)),
        out_shape=jax.ShapeDtypeStruct((_E, _B), jnp.float32),
        scratch_shapes=[pltpu.VMEM((_E, _D), jnp.float32)],
        compiler_params=pltpu.CompilerParams(
            dimension_semantics=("arbitrary",),
        ),
    )(route_vec, expert_embeddings)


@functools.partial(
    pl.kernel,
    mesh=plsc.VectorSubcoreMesh(core_axis_name="c", subcore_axis_name="s"),
    out_type=jax.ShapeDtypeStruct((_B,), jnp.int32),
    scratch_types=[
        pltpu.VMEM((_E, _TPW), jnp.float32),
        pltpu.VMEM((_TPW,), jnp.int32),
    ],
)
def _sc_argmax(scores_hbm, out_hbm, sc_v, idx_v):
    wid = lax.axis_index("s") * _INFO.num_cores + lax.axis_index("c")
    base = wid * _TPW
    pltpu.sync_copy(scores_hbm.at[:, pl.ds(base, _TPW)], sc_v)

    def body(g, carry):
        t0 = g * 16
        m = sc_v[0, pl.ds(t0, 16)]
        bi = jnp.zeros((16,), jnp.int32)
        for e in range(1, _E):
            v = sc_v[e, pl.ds(t0, 16)]
            cmp = v > m
            m = jnp.where(cmp, v, m)
            bi = jnp.where(cmp, jnp.full((16,), e, jnp.int32), bi)
        idx_v[pl.ds(t0, 16)] = bi
        return carry

    lax.fori_loop(0, _TPW // 16, body, 0)
    pltpu.sync_copy(idx_v, out_hbm.at[pl.ds(base, _TPW)])


def kernel(route_vec, expert_embeddings):
    scores_t = _scores_t(route_vec, expert_embeddings)
    idx = _sc_argmax(scores_t)
    return (idx, scores_t.T)


# speedup vs baseline: 1.3076x; 1.3076x over previous
import jax
import jax.numpy as jnp
from jax.experimental import pallas as pl
from jax.experimental.pallas import tpu as pltpu

_B = 32768
_D = 2048
_E = 64
_BLK = 1024


def _router_body(rv_ref, emb_ref, idx_ref, scores_t_ref, w_ref):
    @pl.when(pl.program_id(0) == 0)
    def _prep():
        emb = emb_ref[...]
        norms = jnp.clip(jnp.sqrt(jnp.sum(emb * emb, axis=1, keepdims=True)), 1e-12)
        w_ref[...] = emb / norms

    scores_t = jax.lax.dot_general(
        w_ref[...], rv_ref[...],
        dimension_numbers=(((1,), (1,)), ((), ())),
        preferred_element_type=jnp.float32,
    )
    scores_t_ref[...] = scores_t
    idx_ref[...] = jnp.argmax(scores_t, axis=0).astype(jnp.int32)


def kernel(route_vec, expert_embeddings):
    grid = (_B // _BLK,)
    idx, scores_t = pl.pallas_call(
        _router_body,
        grid=grid,
        in_specs=[
            pl.BlockSpec((_BLK, _D), lambda i: (i, 0)),
            pl.BlockSpec((_E, _D), lambda i: (0, 0)),
        ],
        out_specs=[
            pl.BlockSpec((_BLK,), lambda i: (i,)),
            pl.BlockSpec((_E, _BLK), lambda i: (0, i)),
        ],
        out_shape=[
            jax.ShapeDtypeStruct((_B,), jnp.int32),
            jax.ShapeDtypeStruct((_E, _B), jnp.float32),
        ],
        scratch_shapes=[pltpu.VMEM((_E, _D), jnp.float32)],
        compiler_params=pltpu.CompilerParams(
            dimension_semantics=("arbitrary",),
        ),
    )(route_vec, expert_embeddings)
    return (idx, scores_t.T)
